# Initial kernel scaffold; baseline (speedup 1.0000x reference)
import jax
import jax.numpy as jnp
from jax.experimental import pallas as pl


def kernel(x, edge_index, batch, W1, b1, W2, b2, W3, b3, W4, b4):
    raise NotImplementedError("write your pallas kernel here")


# speedup vs baseline: 9.5488x; 9.5488x over previous
import functools

import jax
import jax.numpy as jnp
from jax import lax
from jax.experimental import pallas as pl
from jax.experimental.pallas import tpu as pltpu
from jax.experimental.pallas import tpu_sc as plsc

N = 10000
E = 320000
G = 128
F0 = 136
D1 = 128
D2 = 128

NC, NS, L = 2, 16, 16
NW = NC * NS
K = 128
C = 79
E_PAD = NW * K * C
N_ACC = 10112
RPT = N_ACC // NS


def _make_sc_scatter(D):
    mesh = plsc.VectorSubcoreMesh(core_axis_name="c", subcore_axis_name="s")

    @functools.partial(
        pl.kernel,
        out_type=jax.ShapeDtypeStruct((NC, N_ACC, D), jnp.float32),
        mesh=mesh,
        scratch_types=[
            pltpu.VMEM((K,), jnp.int32),
            pltpu.VMEM((K,), jnp.int32),
            pltpu.VMEM((K, D), jnp.float32),
            pltpu.VMEM_SHARED((N_ACC, D), jnp.float32),
            pltpu.SemaphoreType.DMA,
        ],
    )
    def sc_scatter(u_hbm, row_hbm, col_hbm, zeros_hbm, out_hbm,
                   ridx, cidx, rows, acc, sem):
        c = lax.axis_index("c")
        s = lax.axis_index("s")
        wid = s * NC + c
        pltpu.sync_copy(zeros_hbm.at[pl.ds(s * RPT, RPT)],
                        acc.at[pl.ds(s * RPT, RPT)])
        plsc.subcore_barrier()
        base0 = wid * (C * K)

        def body(j, carry):
            base = base0 + j * K
            pltpu.sync_copy(row_hbm.at[pl.ds(base, K)], ridx)
            pltpu.sync_copy(col_hbm.at[pl.ds(base, K)], cidx)
            pltpu.async_copy(u_hbm.at[ridx], rows, sem).wait()
            pltpu.sync_copy(rows, acc.at[cidx], add=True)
            return carry

        lax.fori_loop(0, C, body, 0)
        plsc.subcore_barrier()
        pltpu.sync_copy(acc.at[pl.ds(s * RPT, RPT)],
                        out_hbm.at[c, pl.ds(s * RPT, RPT)])

    return sc_scatter


_sc_scatter_d1 = _make_sc_scatter(D1)
_sc_scatter_d2 = _make_sc_scatter(D2)


def _make_sc_degree():
    mesh = plsc.VectorSubcoreMesh(core_axis_name="c", subcore_axis_name="s")

    @functools.partial(
        pl.kernel,
        out_type=jax.ShapeDtypeStruct((NC, N_ACC, D2), jnp.float32),
        mesh=mesh,
        scratch_types=[
            pltpu.VMEM((K,), jnp.int32),
            pltpu.VMEM((K, D2), jnp.float32),
            pltpu.VMEM_SHARED((N_ACC, D2), jnp.float32),
        ],
    )
    def sc_degree(col_hbm, ones_hbm, zeros_hbm, out_hbm, cidx, ones_v, acc):
        c = lax.axis_index("c")
        s = lax.axis_index("s")
        wid = s * NC + c
        pltpu.sync_copy(ones_hbm, ones_v)
        pltpu.sync_copy(zeros_hbm.at[pl.ds(s * RPT, RPT)],
                        acc.at[pl.ds(s * RPT, RPT)])
        plsc.subcore_barrier()
        base0 = wid * (C * K)

        def body(j, carry):
            base = base0 + j * K
            pltpu.sync_copy(col_hbm.at[pl.ds(base, K)], cidx)
            pltpu.sync_copy(ones_v, acc.at[cidx], add=True)
            return carry

        lax.fori_loop(0, C, body, 0)
        plsc.subcore_barrier()
        pltpu.sync_copy(acc.at[pl.ds(s * RPT, RPT)],
                        out_hbm.at[c, pl.ds(s * RPT, RPT)])

    return sc_degree


_sc_degree = _make_sc_degree()


def _tc1_body(degp_ref, x_ref, w1_ref, u_ref, dinv_ref):
    deg = (degp_ref[0, :N, 0:1] + degp_ref[1, :N, 0:1]) + 1.0
    dinv = lax.rsqrt(deg)
    t = jnp.dot(x_ref[...], w1_ref[...], preferred_element_type=jnp.float32)
    u_ref[:N, :] = t * dinv
    u_ref[N:, :] = jnp.zeros((N_ACC - N, D1), jnp.float32)
    dinv_ref[...] = dinv


def _tc2_body(s_ref, u1_ref, dinv_ref, b1_ref, w2_ref, u2_ref):
    dinv = dinv_ref[...]
    agg = s_ref[0, :N, :] + s_ref[1, :N, :] + u1_ref[:N, :]
    h1 = jnp.maximum(dinv * agg + b1_ref[...], 0.0)
    u2 = jnp.dot(h1, w2_ref[...], preferred_element_type=jnp.float32) * dinv
    u2_ref[:N, :] = u2
    u2_ref[N:, :] = jnp.zeros((N_ACC - N, D2), jnp.float32)


def _tc3_body(s_ref, u2_ref, dinv_ref, b2_ref, w34_ref, b34_ref, batch_ref,
              y_ref):
    dinv = dinv_ref[...]
    agg = s_ref[0, :N, :] + s_ref[1, :N, :] + u2_ref[:N, :]
    h2 = jnp.maximum(dinv * agg + b2_ref[...], 0.0)
    z = jnp.sum(h2 * w34_ref[...], axis=1, keepdims=True) + b34_ref[0, 0]
    gid = lax.broadcasted_iota(jnp.int32, (G, N), 0)
    m = (gid == batch_ref[...]).astype(jnp.float32)
    p = jnp.dot(m, z, preferred_element_type=jnp.float32)
    y_ref[...] = jax.nn.sigmoid(p)


def kernel(x, edge_index, batch, W1, b1, W2, b2, W3, b3, W4, b4):
    f32 = jnp.float32
    row = edge_index[0].astype(jnp.int32)
    col = edge_index[1].astype(jnp.int32)
    pad = E_PAD - E
    row_p = jnp.concatenate([row, jnp.zeros((pad,), jnp.int32)])
    col_p = jnp.concatenate([col, jnp.full((pad,), N, jnp.int32)])

    w1p = jnp.pad(W1.astype(f32), ((0, 0), (0, D1 - W1.shape[1])))
    b1p = jnp.pad(b1.astype(f32), (0, D1 - b1.shape[0])).reshape(1, D1)
    w2p = jnp.pad(W2.astype(f32), ((0, D1 - W2.shape[0]), (0, D2 - W2.shape[1])))
    b2p = jnp.pad(b2.astype(f32), (0, D2 - b2.shape[0])).reshape(1, D2)
    w34 = (W3 @ W4).astype(f32)
    b34 = (b3 @ W4 + b4).astype(f32).reshape(1, 1)
    w34p = jnp.pad(w34.reshape(-1), (0, D2 - w34.shape[0])).reshape(1, D2)
    batch2d = batch.astype(jnp.int32).reshape(1, N)

    zeros_d1 = jnp.zeros((N_ACC, D1), f32)
    zeros_d2 = jnp.zeros((N_ACC, D2), f32)
    ones_k = jnp.ones((K, D2), f32)

    deg_p = _sc_degree(col_p, ones_k, zeros_d2)

    u1, dinv = pl.pallas_call(
        _tc1_body,
        out_shape=(jax.ShapeDtypeStruct((N_ACC, D1), f32),
                   jax.ShapeDtypeStruct((N, 1), f32)),
    )(deg_p, x.astype(f32), w1p)

    s1 = _sc_scatter_d1(u1, row_p, col_p, zeros_d1)

    u2 = pl.pallas_call(
        _tc2_body,
        out_shape=jax.ShapeDtypeStruct((N_ACC, D2), f32),
    )(s1, u1, dinv, b1p, w2p)

    s2 = _sc_scatter_d2(u2, row_p, col_p, zeros_d2)

    y = pl.pallas_call(
        _tc3_body,
        out_shape=jax.ShapeDtypeStruct((G, 1), f32),
    )(s2, u2, dinv, b2p, w34p, b34, batch2d)

    return y.reshape(-1)
